# baseline (speedup 1.0000x reference)
import functools

import jax
import jax.numpy as jnp
from jax import lax
from jax.experimental import pallas as pl
from jax.experimental.pallas import tpu as pltpu
from jax.experimental.pallas import tpu_sc as plsc

N = 10000
E = 20000
K = 8
H = 128
DEPTH = 10

NW = 32
EP = 20480
EW = EP // NW
ECH = 16
ET = EW // ECH

_MESH = plsc.VectorSubcoreMesh(core_axis_name="c", subcore_axis_name="s",
                               num_cores=2, num_subcores=16)


def _wid():
    return lax.axis_index("s") * 2 + lax.axis_index("c")


def _sc_row_gather(table, idx, chunk):
    ni, d = idx.shape[0], table.shape[1]
    nchunks = ni // chunk
    per_w = -(-nchunks // NW)

    @functools.partial(
        pl.kernel,
        mesh=_MESH,
        out_type=jax.ShapeDtypeStruct((ni, d), jnp.float32),
        scratch_types=[
            pltpu.VMEM((chunk,), jnp.int32),
            pltpu.VMEM((chunk,), jnp.int32),
            pltpu.VMEM((chunk, d), jnp.float32),
            pltpu.VMEM((chunk, d), jnp.float32),
            pltpu.SemaphoreType.DMA,
            pltpu.SemaphoreType.DMA,
        ],
    )
    def k(table_h, idx_h, out_h, idx0, idx1, rows0, rows1, sem0, sem1):
        w = _wid()
        bufs = ((idx0, rows0, sem0), (idx1, rows1, sem1))

        def start(j, slot):
            iv, rv, sm = bufs[slot]
            c = w + NW * j

            @pl.when(c < nchunks)
            def _():
                pltpu.sync_copy(idx_h.at[pl.ds(c * chunk, chunk)], iv)
                pltpu.async_copy(table_h.at[iv], rv, sm)

        def finish(j, slot):
            iv, rv, sm = bufs[slot]
            c = w + NW * j

            @pl.when(c < nchunks)
            def _():
                pltpu.make_async_copy(table_h.at[iv], rv, sm).wait()
                pltpu.sync_copy(rv, out_h.at[pl.ds(c * chunk, chunk)])

        start(0, 0)
        for j in range(per_w):
            if j + 1 < per_w:
                start(j + 1, (j + 1) % 2)
            finish(j, j % 2)

    return k(table, idx)


NSTREAM = 2
RPS = ECH * K // NSTREAM


@functools.partial(
    pl.kernel,
    mesh=_MESH,
    out_type=jax.ShapeDtypeStruct((EP, 2 * H), jnp.float32),
    scratch_types=[
        pltpu.VMEM((EW * K,), jnp.int32),
        pltpu.VMEM((2, ECH * K, 2 * H), jnp.float32),
        pltpu.VMEM((2, ECH, H), jnp.float32),
        pltpu.VMEM((2, ECH, 2 * H), jnp.float32),
        pltpu.SemaphoreType.DMA,
        pltpu.SemaphoreType.DMA,
        pltpu.SemaphoreType.DMA,
        pltpu.SemaphoreType.DMA,
    ],
)
def _sc_edge_sums(hcat_h, r1b_h, mg_h, out_h,
                  idx_v, rows_v, r1b_v, out_v, gsem0, gsem1, osem0, osem1):
    w = _wid()
    base = w * EW
    gsems = (gsem0, gsem1)
    osems = (osem0, osem1)
    nv = H // 16

    pltpu.sync_copy(mg_h.at[pl.ds(base * K, EW * K)], idx_v)

    def gathers(t, slot):
        gs = []
        for s in range(NSTREAM):
            gs.append(pltpu.make_async_copy(
                hcat_h.at[idx_v.at[pl.ds(t * ECH * K + s * RPS, RPS)]],
                rows_v.at[slot, pl.ds(s * RPS, RPS)],
                gsems[slot]))
        gs.append(pltpu.make_async_copy(
            r1b_h.at[pl.ds(base + t * ECH, ECH)],
            r1b_v.at[slot], gsems[slot]))
        return gs

    def ostore(t, slot):
        return pltpu.make_async_copy(
            out_v.at[slot], out_h.at[pl.ds(base + t * ECH, ECH)],
            osems[slot])

    def start(t, slot):
        for g in gathers(t, slot):
            g.start()

    def work(t, slot):
        for g in gathers(t, slot):
            g.wait()
        rv = rows_v.at[slot]
        r1v = r1b_v.at[slot]
        ov = out_v.at[slot]

        @plsc.parallel_loop(0, ECH, 1)
        def ebody(e):
            r1 = [r1v[e, pl.ds(v * 16, 16)] for v in range(nv)]
            acc_h = [jnp.zeros((16,), jnp.float32) for _ in range(nv)]
            acc_g = [jnp.zeros((16,), jnp.float32) for _ in range(nv)]
            for kk in range(K):
                row = e * K + kk
                hu = [rv[row, pl.ds(H + v * 16, 16)] for v in range(nv)]
                hv = [rv[row, pl.ds(v * 16, 16)] for v in range(nv)]
                u = [jnp.exp(jnp.minimum(-(r1[v] + hu[v]), 80.0))
                     for v in range(nv)]
                d = [1.0 + u[v] for v in range(nv)]
                magic = jnp.full((16,), 0x7EF127EA, jnp.int32)
                y = [lax.bitcast_convert_type(
                        magic - lax.bitcast_convert_type(d[v], jnp.int32),
                        jnp.float32) for v in range(nv)]
                y = [y[v] * (2.0 - d[v] * y[v]) for v in range(nv)]
                y = [y[v] * (2.0 - d[v] * y[v]) for v in range(nv)]
                acc_h = [acc_h[v] + hv[v] for v in range(nv)]
                acc_g = [acc_g[v] + y[v] * hv[v] for v in range(nv)]
            for v in range(nv):
                ov[e, pl.ds(v * 16, 16)] = acc_h[v]
                ov[e, pl.ds(H + v * 16, 16)] = acc_g[v]

        ostore(t, slot).start()

    start(0, 0)

    def outer(i, carry):
        for b in range(2):
            t = 2 * i + b

            @pl.when(t + 1 < ET)
            def _():
                start(t + 1, 1 - b)

            @pl.when(t >= 2)
            def _():
                ostore(t - 2, b).wait()

            work(t, b)
        return carry

    lax.fori_loop(0, ET // 2, outer, 0)
    ostore(ET - 2, 0).wait()
    ostore(ET - 1, 1).wait()


NCH = 16
NCHUNKS = N // NCH
NPW = -(-NCHUNKS // NW)


@functools.partial(
    pl.kernel,
    mesh=_MESH,
    out_type=jax.ShapeDtypeStruct((N, H), jnp.float32),
    scratch_types=[
        pltpu.VMEM((NCH * K,), jnp.int32),
        pltpu.VMEM((NCH * K,), jnp.int32),
        pltpu.VMEM((NCH * K, H), jnp.float32),
        pltpu.VMEM((NCH * K, H), jnp.float32),
        pltpu.VMEM((NCH, H), jnp.float32),
        pltpu.SemaphoreType.DMA,
        pltpu.SemaphoreType.DMA,
    ],
)
def _sc_node_sums(msg_h, ng_h, out_h, idx0, idx1, rows0, rows1, out_v,
                  sem0, sem1):
    w = _wid()
    bufs = ((idx0, rows0, sem0), (idx1, rows1, sem1))
    nv = H // 16

    def start(j, slot):
        iv, rv, sm = bufs[slot]
        c = w + NW * j

        @pl.when(c < NCHUNKS)
        def _():
            pltpu.sync_copy(ng_h.at[pl.ds(c * NCH * K, NCH * K)], iv)
            pltpu.async_copy(msg_h.at[iv], rv, sm)

    def work(j, slot):
        iv, rv, sm = bufs[slot]
        c = w + NW * j

        @pl.when(c < NCHUNKS)
        def _():
            pltpu.make_async_copy(msg_h.at[iv], rv, sm).wait()

            @plsc.parallel_loop(0, NCH, 1, unroll=2)
            def nbody(e):
                acc = [jnp.zeros((16,), jnp.float32) for _ in range(nv)]
                for kk in range(K):
                    row = e * K + kk
                    for v in range(nv):
                        acc[v] = acc[v] + rv[row, pl.ds(v * 16, 16)]
                for v in range(nv):
                    out_v[e, pl.ds(v * 16, 16)] = acc[v]
            pltpu.sync_copy(out_v, out_h.at[pl.ds(c * NCH, NCH)])

    start(0, 0)

    def outer(i, carry):
        for b in range(2):
            j = 2 * i + b

            @pl.when(j + 1 < NPW)
            def _():
                start(j + 1, 1 - b)

            work(j, b)
        return carry

    lax.fori_loop(0, NPW // 2, outer, 0)


EBLK = 640
EGRID = EP // EBLK
NBLK = 400
NGRID = N // NBLK


def _rowmask(blk, i):
    rid = lax.broadcasted_iota(jnp.int32, (blk, 1), 0) + i * blk
    return rid != 0


def _tc_pre_body(x_ref, w_ref, b_ref, cu_ref, pre2_ref, r1b_ref, hcat_ref):
    xb = x_ref[...]
    pre = jnp.dot(xb, w_ref[...], preferred_element_type=jnp.float32) + b_ref[...]
    pre2_ref[...] = pre[:, : 2 * H]
    r1b_ref[...] = pre[:, 2 * H:]
    h1 = jnp.tanh(pre[:, H: 2 * H]) * jax.nn.sigmoid(pre[:, :H])
    h1 = jnp.where(_rowmask(EBLK, pl.program_id(0)), h1, 0.0)
    hcat_ref[:, :H] = h1
    hcat_ref[:, H:] = jnp.dot(h1, cu_ref[...], preferred_element_type=jnp.float32)


def _tc_pre(x, wpre, bpre, cu):
    return pl.pallas_call(
        _tc_pre_body,
        grid=(EGRID,),
        in_specs=[
            pl.BlockSpec((EBLK, H), lambda i: (i, 0)),
            pl.BlockSpec((H, 3 * H), lambda i: (0, 0)),
            pl.BlockSpec((1, 3 * H), lambda i: (0, 0)),
            pl.BlockSpec((H, H), lambda i: (0, 0)),
        ],
        out_specs=[
            pl.BlockSpec((EBLK, 2 * H), lambda i: (i, 0)),
            pl.BlockSpec((EBLK, H), lambda i: (i, 0)),
            pl.BlockSpec((EBLK, 2 * H), lambda i: (i, 0)),
        ],
        out_shape=[
            jax.ShapeDtypeStruct((EP, 2 * H), jnp.float32),
            jax.ShapeDtypeStruct((EP, H), jnp.float32),
            jax.ShapeDtypeStruct((EP, 2 * H), jnp.float32),
        ],
    )(x, wpre, bpre, cu)


def _tc_update_body(last, sums_ref, pre2_ref, bz_ref, bh_ref, cu_ref, out_ref):
    sh = sums_ref[:, :H]
    sg = sums_ref[:, H:]
    z = jax.nn.sigmoid(pre2_ref[:, :H]
                       + jnp.dot(sh, bz_ref[...], preferred_element_type=jnp.float32))
    ph = jnp.tanh(pre2_ref[:, H:]
                  + jnp.dot(sg, bh_ref[...], preferred_element_type=jnp.float32))
    h = (1.0 - z) * sh + z * ph
    h = jnp.where(_rowmask(EBLK, pl.program_id(0)), h, 0.0)
    if last:
        out_ref[...] = h
    else:
        out_ref[:, :H] = h
        out_ref[:, H:] = jnp.dot(h, cu_ref[...], preferred_element_type=jnp.float32)


def _tc_update(sums, pre2, bz, bh, cu, last):
    width = H if last else 2 * H
    return pl.pallas_call(
        functools.partial(_tc_update_body, last),
        grid=(EGRID,),
        in_specs=[
            pl.BlockSpec((EBLK, 2 * H), lambda i: (i, 0)),
            pl.BlockSpec((EBLK, 2 * H), lambda i: (i, 0)),
            pl.BlockSpec((H, H), lambda i: (0, 0)),
            pl.BlockSpec((H, H), lambda i: (0, 0)),
            pl.BlockSpec((H, H), lambda i: (0, 0)),
        ],
        out_specs=pl.BlockSpec((EBLK, width), lambda i: (i, 0)),
        out_shape=jax.ShapeDtypeStruct((EP, width), jnp.float32),
    )(sums, pre2, bz, bh, cu)


def _tc_readout_body(fe_ref, mn_ref, o_ref, b_ref, out_ref):
    cat = jnp.concatenate([fe_ref[...], mn_ref[...]], axis=1)
    nv = jnp.dot(cat, o_ref[...], preferred_element_type=jnp.float32) + b_ref[...]
    out_ref[...] = jnp.maximum(nv, 0.0)


def _tc_readout(fe, mn, ocat, ob):
    return pl.pallas_call(
        _tc_readout_body,
        grid=(NGRID,),
        in_specs=[
            pl.BlockSpec((NBLK, H), lambda i: (i, 0)),
            pl.BlockSpec((NBLK, H), lambda i: (i, 0)),
            pl.BlockSpec((2 * H, H), lambda i: (0, 0)),
            pl.BlockSpec((1, H), lambda i: (0, 0)),
        ],
        out_specs=pl.BlockSpec((NBLK, H), lambda i: (i, 0)),
        out_shape=jax.ShapeDtypeStruct((N, H), jnp.float32),
    )(fe, mn, ocat, ob)


def kernel(fnode, fmess, node_graph, mess_graph, scope, emb,
           Wz_w, Wz_b, Wr_w, Ur_w, Ur_b, Wh_w, Wh_b, out_w, out_b):
    f32 = jnp.float32
    fnode = fnode.astype(jnp.int32)
    fmess = fmess.astype(jnp.int32)

    wpre = jnp.concatenate(
        [Wz_w[:, :H].T, Wh_w[:, :H].T, Wr_w.T], axis=1).astype(f32)
    bpre = jnp.concatenate([Wz_b, Wh_b, Ur_b]).reshape(1, 3 * H).astype(f32)
    bz = Wz_w[:, H:].T.astype(f32)
    bh = Wh_w[:, H:].T.astype(f32)
    cu = Ur_w.T.astype(f32)
    ocat = jnp.concatenate([out_w[:, :H].T, out_w[:, H:].T], axis=0).astype(f32)
    ob = out_b.reshape(1, H).astype(f32)

    fmess_p = jnp.pad(fmess, (0, EP - E))
    mg_flat = jnp.pad(mess_graph.astype(jnp.int32).reshape(-1),
                      (0, EP * K - E * K))
    ng_flat = node_graph.astype(jnp.int32).reshape(-1)
    scope0 = scope[:, 0].astype(jnp.int32)

    fnode_emb = _sc_row_gather(emb.astype(f32), fnode, 80)
    x = _sc_row_gather(fnode_emb, fmess_p, 80)

    pre2, r1b, hcat = _tc_pre(x, wpre, bpre, cu)

    for it in range(1, DEPTH):
        sums = _sc_edge_sums(hcat, r1b, mg_flat)
        hcat = _tc_update(sums, pre2, bz, bh, cu, last=(it == DEPTH - 1))

    messages = hcat[:E]

    mess_nei = _sc_node_sums(messages, ng_flat)
    node_vecs = _tc_readout(fnode_emb, mess_nei, ocat, ob)
    tree_vecs = _sc_row_gather(node_vecs, scope0, 64)

    return tree_vecs, messages


# speedup vs baseline: 1.0193x; 1.0193x over previous
import functools

import jax
import jax.numpy as jnp
from jax import lax
from jax.experimental import pallas as pl
from jax.experimental.pallas import tpu as pltpu
from jax.experimental.pallas import tpu_sc as plsc

N = 10000
E = 20000
K = 8
H = 128
DEPTH = 10

NW = 32
EP = 20480
EW = EP // NW
ECH = 16
ET = EW // ECH

_MESH = plsc.VectorSubcoreMesh(core_axis_name="c", subcore_axis_name="s",
                               num_cores=2, num_subcores=16)


def _wid():
    return lax.axis_index("s") * 2 + lax.axis_index("c")


def _sc_row_gather(table, idx, chunk):
    ni, d = idx.shape[0], table.shape[1]
    nchunks = ni // chunk
    per_w = -(-nchunks // NW)

    @functools.partial(
        pl.kernel,
        mesh=_MESH,
        out_type=jax.ShapeDtypeStruct((ni, d), jnp.float32),
        scratch_types=[
            pltpu.VMEM((chunk,), jnp.int32),
            pltpu.VMEM((chunk,), jnp.int32),
            pltpu.VMEM((chunk, d), jnp.float32),
            pltpu.VMEM((chunk, d), jnp.float32),
            pltpu.SemaphoreType.DMA,
            pltpu.SemaphoreType.DMA,
        ],
    )
    def k(table_h, idx_h, out_h, idx0, idx1, rows0, rows1, sem0, sem1):
        w = _wid()
        bufs = ((idx0, rows0, sem0), (idx1, rows1, sem1))

        def start(j, slot):
            iv, rv, sm = bufs[slot]
            c = w + NW * j

            @pl.when(c < nchunks)
            def _():
                pltpu.sync_copy(idx_h.at[pl.ds(c * chunk, chunk)], iv)
                pltpu.async_copy(table_h.at[iv], rv, sm)

        def finish(j, slot):
            iv, rv, sm = bufs[slot]
            c = w + NW * j

            @pl.when(c < nchunks)
            def _():
                pltpu.make_async_copy(table_h.at[iv], rv, sm).wait()
                pltpu.sync_copy(rv, out_h.at[pl.ds(c * chunk, chunk)])

        start(0, 0)
        for j in range(per_w):
            if j + 1 < per_w:
                start(j + 1, (j + 1) % 2)
            finish(j, j % 2)

    return k(table, idx)


NSTREAM = 2
RPS = ECH * K // NSTREAM


@functools.partial(
    pl.kernel,
    mesh=_MESH,
    out_type=jax.ShapeDtypeStruct((EP, 2 * H), jnp.float32),
    scratch_types=[
        pltpu.VMEM((EW * K,), jnp.int32),
        pltpu.VMEM((2, ECH * K, 2 * H), jnp.float32),
        pltpu.VMEM((2, ECH, H), jnp.float32),
        pltpu.VMEM((2, ECH, 2 * H), jnp.float32),
        pltpu.SemaphoreType.DMA,
        pltpu.SemaphoreType.DMA,
        pltpu.SemaphoreType.DMA,
        pltpu.SemaphoreType.DMA,
    ],
)
def _sc_edge_sums(hcat_h, r1b_h, mg_h, out_h,
                  idx_v, rows_v, r1b_v, out_v, gsem0, gsem1, osem0, osem1):
    w = _wid()
    base = w * EW
    gsems = (gsem0, gsem1)
    osems = (osem0, osem1)
    nv = H // 16

    pltpu.sync_copy(mg_h.at[pl.ds(base * K, EW * K)], idx_v)

    def gathers(t, slot):
        gs = []
        for s in range(NSTREAM):
            gs.append(pltpu.make_async_copy(
                hcat_h.at[idx_v.at[pl.ds(t * ECH * K + s * RPS, RPS)]],
                rows_v.at[slot, pl.ds(s * RPS, RPS)],
                gsems[slot]))
        gs.append(pltpu.make_async_copy(
            r1b_h.at[pl.ds(base + t * ECH, ECH)],
            r1b_v.at[slot], gsems[slot]))
        return gs

    def ostore(t, slot):
        return pltpu.make_async_copy(
            out_v.at[slot], out_h.at[pl.ds(base + t * ECH, ECH)],
            osems[slot])

    def start(t, slot):
        for g in gathers(t, slot):
            g.start()

    def work(t, slot):
        for g in gathers(t, slot):
            g.wait()
        rv = rows_v.at[slot]
        r1v = r1b_v.at[slot]
        ov = out_v.at[slot]

        @plsc.parallel_loop(0, ECH, 1)
        def ebody(e):
            r1 = [r1v[e, pl.ds(v * 16, 16)] for v in range(nv)]
            acc_h = [jnp.zeros((16,), jnp.float32) for _ in range(nv)]
            acc_g = [jnp.zeros((16,), jnp.float32) for _ in range(nv)]
            for kk in range(K):
                row = e * K + kk
                hu = [rv[row, pl.ds(H + v * 16, 16)] for v in range(nv)]
                hv = [rv[row, pl.ds(v * 16, 16)] for v in range(nv)]
                u = [jnp.exp(jnp.minimum(-(r1[v] + hu[v]), 80.0))
                     for v in range(nv)]
                d = [1.0 + u[v] for v in range(nv)]
                magic = jnp.full((16,), 0x7EF127EA, jnp.int32)
                y = [lax.bitcast_convert_type(
                        magic - lax.bitcast_convert_type(d[v], jnp.int32),
                        jnp.float32) for v in range(nv)]
                y = [y[v] * (2.0 - d[v] * y[v]) for v in range(nv)]
                y = [y[v] * (2.0 - d[v] * y[v]) for v in range(nv)]
                y = [y[v] * (2.0 - d[v] * y[v]) for v in range(nv)]
                acc_h = [acc_h[v] + hv[v] for v in range(nv)]
                acc_g = [acc_g[v] + y[v] * hv[v] for v in range(nv)]
            for v in range(nv):
                ov[e, pl.ds(v * 16, 16)] = acc_h[v]
                ov[e, pl.ds(H + v * 16, 16)] = acc_g[v]

        ostore(t, slot).start()

    start(0, 0)

    def outer(i, carry):
        for b in range(2):
            t = 2 * i + b

            @pl.when(t + 1 < ET)
            def _():
                start(t + 1, 1 - b)

            @pl.when(t >= 2)
            def _():
                ostore(t - 2, b).wait()

            work(t, b)
        return carry

    lax.fori_loop(0, ET // 2, outer, 0)
    ostore(ET - 2, 0).wait()
    ostore(ET - 1, 1).wait()


NCH = 16
NCHUNKS = N // NCH
NPW = -(-NCHUNKS // NW)


@functools.partial(
    pl.kernel,
    mesh=_MESH,
    out_type=jax.ShapeDtypeStruct((N, H), jnp.float32),
    scratch_types=[
        pltpu.VMEM((NCH * K,), jnp.int32),
        pltpu.VMEM((NCH * K,), jnp.int32),
        pltpu.VMEM((NCH * K, H), jnp.float32),
        pltpu.VMEM((NCH * K, H), jnp.float32),
        pltpu.VMEM((NCH, H), jnp.float32),
        pltpu.SemaphoreType.DMA,
        pltpu.SemaphoreType.DMA,
    ],
)
def _sc_node_sums(msg_h, ng_h, out_h, idx0, idx1, rows0, rows1, out_v,
                  sem0, sem1):
    w = _wid()
    bufs = ((idx0, rows0, sem0), (idx1, rows1, sem1))
    nv = H // 16

    def start(j, slot):
        iv, rv, sm = bufs[slot]
        c = w + NW * j

        @pl.when(c < NCHUNKS)
        def _():
            pltpu.sync_copy(ng_h.at[pl.ds(c * NCH * K, NCH * K)], iv)
            pltpu.async_copy(msg_h.at[iv], rv, sm)

    def work(j, slot):
        iv, rv, sm = bufs[slot]
        c = w + NW * j

        @pl.when(c < NCHUNKS)
        def _():
            pltpu.make_async_copy(msg_h.at[iv], rv, sm).wait()

            @plsc.parallel_loop(0, NCH, 1, unroll=2)
            def nbody(e):
                acc = [jnp.zeros((16,), jnp.float32) for _ in range(nv)]
                for kk in range(K):
                    row = e * K + kk
                    for v in range(nv):
                        acc[v] = acc[v] + rv[row, pl.ds(v * 16, 16)]
                for v in range(nv):
                    out_v[e, pl.ds(v * 16, 16)] = acc[v]
            pltpu.sync_copy(out_v, out_h.at[pl.ds(c * NCH, NCH)])

    start(0, 0)

    def outer(i, carry):
        for b in range(2):
            j = 2 * i + b

            @pl.when(j + 1 < NPW)
            def _():
                start(j + 1, 1 - b)

            work(j, b)
        return carry

    lax.fori_loop(0, NPW // 2, outer, 0)


EBLK = 640
EGRID = EP // EBLK
NBLK = 400
NGRID = N // NBLK


def _rowmask(blk, i):
    rid = lax.broadcasted_iota(jnp.int32, (blk, 1), 0) + i * blk
    return rid != 0


def _tc_pre_body(x_ref, w_ref, b_ref, cu_ref, pre2_ref, r1b_ref, hcat_ref):
    xb = x_ref[...]
    pre = jnp.dot(xb, w_ref[...], preferred_element_type=jnp.float32) + b_ref[...]
    pre2_ref[...] = pre[:, : 2 * H]
    r1b_ref[...] = pre[:, 2 * H:]
    h1 = jnp.tanh(pre[:, H: 2 * H]) * jax.nn.sigmoid(pre[:, :H])
    h1 = jnp.where(_rowmask(EBLK, pl.program_id(0)), h1, 0.0)
    hcat_ref[:, :H] = h1
    hcat_ref[:, H:] = jnp.dot(h1, cu_ref[...], preferred_element_type=jnp.float32)


def _tc_pre(x, wpre, bpre, cu):
    return pl.pallas_call(
        _tc_pre_body,
        grid=(EGRID,),
        in_specs=[
            pl.BlockSpec((EBLK, H), lambda i: (i, 0)),
            pl.BlockSpec((H, 3 * H), lambda i: (0, 0)),
            pl.BlockSpec((1, 3 * H), lambda i: (0, 0)),
            pl.BlockSpec((H, H), lambda i: (0, 0)),
        ],
        out_specs=[
            pl.BlockSpec((EBLK, 2 * H), lambda i: (i, 0)),
            pl.BlockSpec((EBLK, H), lambda i: (i, 0)),
            pl.BlockSpec((EBLK, 2 * H), lambda i: (i, 0)),
        ],
        out_shape=[
            jax.ShapeDtypeStruct((EP, 2 * H), jnp.float32),
            jax.ShapeDtypeStruct((EP, H), jnp.float32),
            jax.ShapeDtypeStruct((EP, 2 * H), jnp.float32),
        ],
    )(x, wpre, bpre, cu)


def _tc_update_body(last, sums_ref, pre2_ref, bz_ref, bh_ref, cu_ref, out_ref):
    sh = sums_ref[:, :H]
    sg = sums_ref[:, H:]
    z = jax.nn.sigmoid(pre2_ref[:, :H]
                       + jnp.dot(sh, bz_ref[...], preferred_element_type=jnp.float32))
    ph = jnp.tanh(pre2_ref[:, H:]
                  + jnp.dot(sg, bh_ref[...], preferred_element_type=jnp.float32))
    h = (1.0 - z) * sh + z * ph
    h = jnp.where(_rowmask(EBLK, pl.program_id(0)), h, 0.0)
    if last:
        out_ref[...] = h
    else:
        out_ref[:, :H] = h
        out_ref[:, H:] = jnp.dot(h, cu_ref[...], preferred_element_type=jnp.float32)


def _tc_update(sums, pre2, bz, bh, cu, last):
    width = H if last else 2 * H
    return pl.pallas_call(
        functools.partial(_tc_update_body, last),
        grid=(EGRID,),
        in_specs=[
            pl.BlockSpec((EBLK, 2 * H), lambda i: (i, 0)),
            pl.BlockSpec((EBLK, 2 * H), lambda i: (i, 0)),
            pl.BlockSpec((H, H), lambda i: (0, 0)),
            pl.BlockSpec((H, H), lambda i: (0, 0)),
            pl.BlockSpec((H, H), lambda i: (0, 0)),
        ],
        out_specs=pl.BlockSpec((EBLK, width), lambda i: (i, 0)),
        out_shape=jax.ShapeDtypeStruct((EP, width), jnp.float32),
    )(sums, pre2, bz, bh, cu)


def _tc_readout_body(fe_ref, mn_ref, o_ref, b_ref, out_ref):
    cat = jnp.concatenate([fe_ref[...], mn_ref[...]], axis=1)
    nv = jnp.dot(cat, o_ref[...], preferred_element_type=jnp.float32) + b_ref[...]
    out_ref[...] = jnp.maximum(nv, 0.0)


def _tc_readout(fe, mn, ocat, ob):
    return pl.pallas_call(
        _tc_readout_body,
        grid=(NGRID,),
        in_specs=[
            pl.BlockSpec((NBLK, H), lambda i: (i, 0)),
            pl.BlockSpec((NBLK, H), lambda i: (i, 0)),
            pl.BlockSpec((2 * H, H), lambda i: (0, 0)),
            pl.BlockSpec((1, H), lambda i: (0, 0)),
        ],
        out_specs=pl.BlockSpec((NBLK, H), lambda i: (i, 0)),
        out_shape=jax.ShapeDtypeStruct((N, H), jnp.float32),
    )(fe, mn, ocat, ob)


def kernel(fnode, fmess, node_graph, mess_graph, scope, emb,
           Wz_w, Wz_b, Wr_w, Ur_w, Ur_b, Wh_w, Wh_b, out_w, out_b):
    f32 = jnp.float32
    fnode = fnode.astype(jnp.int32)
    fmess = fmess.astype(jnp.int32)

    wpre = jnp.concatenate(
        [Wz_w[:, :H].T, Wh_w[:, :H].T, Wr_w.T], axis=1).astype(f32)
    bpre = jnp.concatenate([Wz_b, Wh_b, Ur_b]).reshape(1, 3 * H).astype(f32)
    bz = Wz_w[:, H:].T.astype(f32)
    bh = Wh_w[:, H:].T.astype(f32)
    cu = Ur_w.T.astype(f32)
    ocat = jnp.concatenate([out_w[:, :H].T, out_w[:, H:].T], axis=0).astype(f32)
    ob = out_b.reshape(1, H).astype(f32)

    fmess_p = jnp.pad(fmess, (0, EP - E))
    mg_flat = jnp.pad(mess_graph.astype(jnp.int32).reshape(-1),
                      (0, EP * K - E * K))
    ng_flat = node_graph.astype(jnp.int32).reshape(-1)
    scope0 = scope[:, 0].astype(jnp.int32)

    fnode_emb = _sc_row_gather(emb.astype(f32), fnode, 80)
    x = _sc_row_gather(fnode_emb, fmess_p, 80)

    pre2, r1b, hcat = _tc_pre(x, wpre, bpre, cu)

    for it in range(1, DEPTH):
        sums = _sc_edge_sums(hcat, r1b, mg_flat)
        hcat = _tc_update(sums, pre2, bz, bh, cu, last=(it == DEPTH - 1))

    messages = hcat[:E]

    mess_nei = _sc_node_sums(messages, ng_flat)
    node_vecs = _tc_readout(fnode_emb, mess_nei, ocat, ob)
    tree_vecs = _sc_row_gather(node_vecs, scope0, 64)

    return tree_vecs, messages
